# Initial kernel scaffold; baseline (speedup 1.0000x reference)
import functools
import jax
import jax.numpy as jnp
from jax import lax
from jax.experimental import pallas as pl
from jax.experimental.pallas import tpu as pltpu
from jax.experimental.pallas import tpu_sc as plsc

N_NODES = 10000
DIM = 128
NUM_GRAPHS = 64

NC, NS, L = 2, 16, 16
NW = NC * NS
NPAD = 10240
W144 = 144
K = 48
ROWS_PER_TILE = NPAD // NS


def _leaky(x, slope):
  return jnp.where(x > 0, x, x * slope)


def _lin_body(x_ref, w_ref, as_ref, ad_ref, h_ref, s_ref, d_ref, m_ref):
  x = x_ref[...]
  h = lax.dot_general(x, w_ref[...], (((1,), (1,)), ((), ())),
                      preferred_element_type=jnp.float32)
  h_ref[...] = h
  s = lax.dot_general(h, as_ref[...], (((1,), (0,)), ((), ())),
                      preferred_element_type=jnp.float32)
  d = lax.dot_general(h, ad_ref[...], (((1,), (0,)), ((), ())),
                      preferred_element_type=jnp.float32)
  s_ref[...] = s
  d_ref[...] = d
  m_ref[0, 0] = _leaky(jnp.max(s) + jnp.max(d), 0.2)


def _tc_lin(x, w, att_s, att_d):
  return pl.pallas_call(
      _lin_body,
      out_shape=[
          jax.ShapeDtypeStruct((NPAD, DIM), jnp.float32),
          jax.ShapeDtypeStruct((NPAD, 1), jnp.float32),
          jax.ShapeDtypeStruct((NPAD, 1), jnp.float32),
          jax.ShapeDtypeStruct((1, 1), jnp.float32),
      ],
  )(x, w, att_s.reshape(DIM, 1), att_d.reshape(DIM, 1))


def _mid_body(a0_ref, a1_ref, b_ref, w_ref, as_ref, ad_ref,
              h_ref, s_ref, d_ref, m_ref):
  a0 = a0_ref[...]
  a1 = a1_ref[...]
  num = a0[:, :DIM] + a1[:, :DIM]
  den = a0[:, DIM:DIM + 1] + a1[:, DIM:DIM + 1]
  o = num / (den + 1e-16) + b_ref[...]
  o = _leaky(o, 0.01)
  h = lax.dot_general(o, w_ref[...], (((1,), (1,)), ((), ())),
                      preferred_element_type=jnp.float32)
  h_ref[...] = h
  s = lax.dot_general(h, as_ref[...], (((1,), (0,)), ((), ())),
                      preferred_element_type=jnp.float32)
  d = lax.dot_general(h, ad_ref[...], (((1,), (0,)), ((), ())),
                      preferred_element_type=jnp.float32)
  s_ref[...] = s
  d_ref[...] = d
  m_ref[0, 0] = _leaky(jnp.max(s) + jnp.max(d), 0.2)


def _tc_mid(a0, a1, b, w, att_s, att_d):
  return pl.pallas_call(
      _mid_body,
      out_shape=[
          jax.ShapeDtypeStruct((NPAD, DIM), jnp.float32),
          jax.ShapeDtypeStruct((NPAD, 1), jnp.float32),
          jax.ShapeDtypeStruct((NPAD, 1), jnp.float32),
          jax.ShapeDtypeStruct((1, 1), jnp.float32),
      ],
  )(a0, a1, b.reshape(1, DIM), w, att_s.reshape(DIM, 1), att_d.reshape(DIM, 1))


def _pool_body(a0_ref, a1_ref, b_ref, batch_ref, out_ref):
  a0 = a0_ref[...]
  a1 = a1_ref[...]
  num = a0[:, :DIM] + a1[:, :DIM]
  den = a0[:, DIM:DIM + 1] + a1[:, DIM:DIM + 1]
  o = num / (den + 1e-16) + b_ref[...]
  o = o[:N_NODES]
  bb = batch_ref[...]
  iot = lax.broadcasted_iota(jnp.int32, (NUM_GRAPHS, N_NODES), 0)
  onehot = (iot == bb).astype(jnp.float32)
  out_ref[...] = lax.dot_general(onehot, o, (((1,), (0,)), ((), ())),
                                 preferred_element_type=jnp.float32)


def _tc_pool(a0, a1, b, batch):
  return pl.pallas_call(
      _pool_body,
      out_shape=jax.ShapeDtypeStruct((NUM_GRAPHS, DIM), jnp.float32),
  )(a0, a1, b.reshape(1, DIM), batch.reshape(1, N_NODES))


def _sc_edge_body(nch,
                  h_hbm, asrc_hbm, adst_hbm, src_hbm, dst_hbm, mvec_hbm,
                  zeros_hbm, out_hbm,
                  acc_s, asrc_v, adst_v, src_v, dst_v, p_v, g_v, rows_v, m_v):
  cid = lax.axis_index("c")
  sid = lax.axis_index("s")
  wid = sid * NC + cid

  rbase = sid * ROWS_PER_TILE
  pltpu.sync_copy(zeros_hbm.at[pl.ds(rbase, ROWS_PER_TILE)],
                  acc_s.at[pl.ds(rbase, ROWS_PER_TILE)])

  pltpu.sync_copy(asrc_hbm, asrc_v)
  pltpu.sync_copy(adst_hbm, adst_v)
  pltpu.sync_copy(src_hbm.at[wid], src_v)
  pltpu.sync_copy(dst_hbm.at[wid], dst_v)
  pltpu.sync_copy(mvec_hbm, m_v)
  mv = m_v[...]
  m0 = (lax.iota(jnp.int32, (L,)) == 0).astype(jnp.float32)

  @pl.loop(0, nch)
  def _p_loop(c):
    for l in range(K // L):
      si = src_v[c, pl.ds(l * L, L)]
      di = dst_v[c, pl.ds(l * L, L)]
      a = plsc.load_gather(asrc_v, [si]) + plsc.load_gather(adst_v, [di])
      a = _leaky(a, 0.2)
      p_v[c, pl.ds(l * L, L)] = jnp.exp(a - mv)

  plsc.subcore_barrier()

  @pl.loop(0, nch)
  def _e_loop(c):
    pltpu.sync_copy(h_hbm.at[src_v.at[c]], g_v)
    for e in range(K):
      pb = plsc.load_gather(p_v, [jnp.full((L,), c, jnp.int32),
                                  jnp.full((L,), e, jnp.int32)])
      for j in range(DIM // L):
        rows_v[e, pl.ds(j * L, L)] = g_v[e, pl.ds(j * L, L)] * pb
      rows_v[e, pl.ds(DIM, L)] = pb * m0
    pltpu.sync_copy(rows_v, acc_s.at[dst_v.at[c]], add=True)

  plsc.subcore_barrier()

  pltpu.sync_copy(acc_s.at[pl.ds(rbase, ROWS_PER_TILE)],
                  out_hbm.at[cid, pl.ds(rbase, ROWS_PER_TILE)])


def _sc_edge(h, asrc, adst, src3, dst3, mvec, zeros):
  nch = src3.shape[1]
  kern = pl.kernel(
      functools.partial(_sc_edge_body, nch),
      out_type=jax.ShapeDtypeStruct((NC, NPAD, W144), jnp.float32),
      mesh=plsc.VectorSubcoreMesh(core_axis_name="c", subcore_axis_name="s",
                                  num_cores=NC, num_subcores=NS),
      scratch_types=[
          pltpu.VMEM_SHARED((NPAD, W144), jnp.float32),
          pltpu.VMEM((NPAD,), jnp.float32),
          pltpu.VMEM((NPAD,), jnp.float32),
          pltpu.VMEM((nch, K), jnp.int32),
          pltpu.VMEM((nch, K), jnp.int32),
          pltpu.VMEM((nch, K), jnp.float32),
          pltpu.VMEM((K, DIM), jnp.float32),
          pltpu.VMEM((K, W144), jnp.float32),
          pltpu.VMEM((L,), jnp.float32),
      ],
  )
  return kern(h, asrc, adst, src3, dst3, mvec, zeros)


def kernel(x, edge_index, batch, W1, att_src1, att_dst1, b1,
           W2, att_src2, att_dst2, b2):
  n = x.shape[0]
  e = edge_index.shape[1]
  ei = edge_index.astype(jnp.int32)

  etot = ((e + n + NW * K - 1) // (NW * K)) * (NW * K)
  loop_idx = jnp.arange(n, dtype=jnp.int32)
  pad = jnp.full((etot - e - n,), N_NODES, jnp.int32)
  src3 = jnp.concatenate([ei[0], loop_idx, pad]).reshape(NW, -1, K)
  dst3 = jnp.concatenate([ei[1], loop_idx, pad]).reshape(NW, -1, K)

  x_pad = jnp.zeros((NPAD, DIM), jnp.float32).at[:n].set(x)
  zeros = jnp.zeros((NPAD, W144), jnp.float32)

  h1, s1, d1, m1 = _tc_lin(x_pad, W1, att_src1, att_dst1)
  mvec1 = jnp.full((L,), m1[0, 0], jnp.float32)
  acc1 = _sc_edge(h1, s1.reshape(NPAD), d1.reshape(NPAD), src3, dst3,
                  mvec1, zeros)

  h2, s2, d2, m2 = _tc_mid(acc1[0], acc1[1], b1, W2, att_src2, att_dst2)
  mvec2 = jnp.full((L,), m2[0, 0], jnp.float32)
  acc2 = _sc_edge(h2, s2.reshape(NPAD), d2.reshape(NPAD), src3, dst3,
                  mvec2, zeros)

  return _tc_pool(acc2[0], acc2[1], b2, batch.astype(jnp.int32))


# speedup vs baseline: 17.2835x; 17.2835x over previous
import functools
import jax
import jax.numpy as jnp
from jax import lax
from jax.experimental import pallas as pl
from jax.experimental.pallas import tpu as pltpu
from jax.experimental.pallas import tpu_sc as plsc

N_NODES = 10000
DIM = 128
NUM_GRAPHS = 64

NC, NS, L = 2, 16, 16
NW = NC * NS
NPAD = 10112
K = 48
HSZ = 4096
RPT = NPAD // NS


def _leaky(x, slope):
  return jnp.where(x > 0, x, x * slope)


def _lin_body(x_ref, w_ref, as_ref, ad_ref, h_ref, s_ref, d_ref, m_ref):
  x = x_ref[...]
  h = lax.dot_general(x, w_ref[...], (((1,), (1,)), ((), ())),
                      preferred_element_type=jnp.float32)
  h_ref[...] = h
  s = lax.dot_general(h, as_ref[...], (((1,), (0,)), ((), ())),
                      preferred_element_type=jnp.float32)
  d = lax.dot_general(h, ad_ref[...], (((1,), (0,)), ((), ())),
                      preferred_element_type=jnp.float32)
  rows = lax.broadcasted_iota(jnp.int32, (NPAD, 1), 0)
  s = jnp.where(rows < N_NODES, s, 0.0)
  d = jnp.where(rows < N_NODES, d, 0.0)
  s_ref[...] = s
  d_ref[...] = d
  m_ref[...] = _leaky(jnp.max(s) + jnp.max(d), 0.2).reshape(1, 1)


def _tc_lin(x, w, att_s, att_d):
  return pl.pallas_call(
      _lin_body,
      out_shape=[
          jax.ShapeDtypeStruct((NPAD, DIM), jnp.float32),
          jax.ShapeDtypeStruct((NPAD, 1), jnp.float32),
          jax.ShapeDtypeStruct((NPAD, 1), jnp.float32),
          jax.ShapeDtypeStruct((1, 1), jnp.float32),
      ],
  )(x, w, att_s.reshape(DIM, 1), att_d.reshape(DIM, 1))


def _mid_body(a0_ref, a1_ref, den_ref, b_ref, w_ref, as_ref, ad_ref,
              h_ref, s_ref, d_ref, m_ref):
  num = a0_ref[...] + a1_ref[...]
  den = jnp.sum(den_ref[...], axis=0).reshape(NPAD, 1)
  o = num / (den + 1e-16) + b_ref[...]
  o = _leaky(o, 0.01)
  h = lax.dot_general(o, w_ref[...], (((1,), (1,)), ((), ())),
                      preferred_element_type=jnp.float32)
  h = jnp.where(lax.broadcasted_iota(jnp.int32, (NPAD, 1), 0) < N_NODES,
                h, 0.0)
  h_ref[...] = h
  s = lax.dot_general(h, as_ref[...], (((1,), (0,)), ((), ())),
                      preferred_element_type=jnp.float32)
  d = lax.dot_general(h, ad_ref[...], (((1,), (0,)), ((), ())),
                      preferred_element_type=jnp.float32)
  rows = lax.broadcasted_iota(jnp.int32, (NPAD, 1), 0)
  s = jnp.where(rows < N_NODES, s, 0.0)
  d = jnp.where(rows < N_NODES, d, 0.0)
  s_ref[...] = s
  d_ref[...] = d
  m_ref[...] = _leaky(jnp.max(s) + jnp.max(d), 0.2).reshape(1, 1)


def _tc_mid(a0, a1, den, b, w, att_s, att_d):
  return pl.pallas_call(
      _mid_body,
      out_shape=[
          jax.ShapeDtypeStruct((NPAD, DIM), jnp.float32),
          jax.ShapeDtypeStruct((NPAD, 1), jnp.float32),
          jax.ShapeDtypeStruct((NPAD, 1), jnp.float32),
          jax.ShapeDtypeStruct((1, 1), jnp.float32),
      ],
  )(a0, a1, den, b.reshape(1, DIM), w,
    att_s.reshape(DIM, 1), att_d.reshape(DIM, 1))


def _pool_body(a0_ref, a1_ref, den_ref, b_ref, batch_ref, out_ref):
  num = a0_ref[...] + a1_ref[...]
  den = jnp.sum(den_ref[...], axis=0).reshape(NPAD, 1)
  o = num / (den + 1e-16) + b_ref[...]
  o = o[:N_NODES]
  bb = batch_ref[...]
  iot = lax.broadcasted_iota(jnp.int32, (NUM_GRAPHS, N_NODES), 0)
  onehot = (iot == bb).astype(jnp.float32)
  out_ref[...] = lax.dot_general(onehot, o, (((1,), (0,)), ((), ())),
                                 preferred_element_type=jnp.float32)


def _tc_pool(a0, a1, den, b, batch):
  return pl.pallas_call(
      _pool_body,
      out_shape=jax.ShapeDtypeStruct((NUM_GRAPHS, DIM), jnp.float32),
  )(a0, a1, den, b.reshape(1, DIM), batch.reshape(1, N_NODES))


def _sc_edge_body(nch,
                  h_hbm, asrc_hbm, adst_hbm, src_hbm, dst_hbm, mvec_hbm,
                  zeros_hbm, acc_out, den_out,
                  acc_s, asrc_v, adst_v, den_v,
                  srcc0_v, dstc0_v, srcc1_v, dstc1_v, g0_v, g1_v, m_v):
  cid = lax.axis_index("c")
  sid = lax.axis_index("s")
  wid = sid * NC + cid

  rbase = sid * RPT
  pltpu.sync_copy(zeros_hbm.at[pl.ds(rbase, RPT)],
                  acc_s.at[pl.ds(rbase, RPT)])

  @pl.loop(0, NPAD // L)
  def _z_loop(c):
    den_v[pl.ds(c * L, L)] = jnp.zeros((L,), jnp.float32)

  pltpu.sync_copy(asrc_hbm, asrc_v)
  pltpu.sync_copy(adst_hbm, adst_v)
  pltpu.sync_copy(mvec_hbm, m_v)

  plsc.subcore_barrier()

  def _chunk(c, srcc_v, dstc_v, g_v):
    pltpu.sync_copy(src_hbm.at[wid, c], srcc_v)
    pltpu.sync_copy(dst_hbm.at[wid, c], dstc_v)
    pltpu.sync_copy(h_hbm.at[srcc_v], g_v)
    mv = m_v[...]
    iot = lax.iota(jnp.int32, L)
    for l in range(K // L):
      si = srcc_v[pl.ds(l * L, L)]
      di = dstc_v[pl.ds(l * L, L)]
      a = plsc.load_gather(asrc_v, [si]) + plsc.load_gather(adst_v, [di])
      p = jnp.exp(_leaky(a, 0.2) - mv)
      for i in range(L):
        plsc.addupdate_scatter(den_v, [di], p, mask=iot == i)
      for i in range(L):
        e = l * L + i
        pb = lax.gather(
            p, jnp.full((L, 1), i, jnp.int32),
            lax.GatherDimensionNumbers(offset_dims=(),
                                       collapsed_slice_dims=(0,),
                                       start_index_map=(0,)),
            (1,), mode=lax.GatherScatterMode.PROMISE_IN_BOUNDS)
        for j in range(DIM // L):
          g_v[e, pl.ds(j * L, L)] = g_v[e, pl.ds(j * L, L)] * pb
    pltpu.sync_copy(g_v, acc_s.at[dstc_v], add=True)

  @pl.loop(0, nch // 2)
  def _e_loop(cc):
    _chunk(cc * 2, srcc0_v, dstc0_v, g0_v)
    _chunk(cc * 2 + 1, srcc1_v, dstc1_v, g1_v)

  plsc.subcore_barrier()

  pltpu.sync_copy(acc_s.at[pl.ds(rbase, RPT)],
                  acc_out.at[cid, pl.ds(rbase, RPT)])
  pltpu.sync_copy(den_v, den_out.at[wid])


def _sc_edge(h, asrc, adst, src3, dst3, mvec, zeros):
  nch = src3.shape[1]
  kern = pl.kernel(
      functools.partial(_sc_edge_body, nch),
      out_type=[
          jax.ShapeDtypeStruct((NC, NPAD, DIM), jnp.float32),
          jax.ShapeDtypeStruct((NW, NPAD), jnp.float32),
      ],
      mesh=plsc.VectorSubcoreMesh(core_axis_name="c", subcore_axis_name="s",
                                  num_cores=NC, num_subcores=NS),
      compiler_params=pltpu.CompilerParams(needs_layout_passes=False),
      scratch_types=[
          pltpu.VMEM_SHARED((NPAD, DIM), jnp.float32),
          pltpu.VMEM((NPAD,), jnp.float32),
          pltpu.VMEM((NPAD,), jnp.float32),
          pltpu.VMEM((NPAD,), jnp.float32),
          pltpu.VMEM((K,), jnp.int32),
          pltpu.VMEM((K,), jnp.int32),
          pltpu.VMEM((K,), jnp.int32),
          pltpu.VMEM((K,), jnp.int32),
          pltpu.VMEM((K, DIM), jnp.float32),
          pltpu.VMEM((K, DIM), jnp.float32),
          pltpu.VMEM((L,), jnp.float32),
      ],
  )
  return kern(h, asrc, adst, src3, dst3, mvec, zeros)


def kernel(x, edge_index, batch, W1, att_src1, att_dst1, b1,
           W2, att_src2, att_dst2, b2):
  n = x.shape[0]
  e = edge_index.shape[1]
  ei = edge_index.astype(jnp.int32)

  etot = ((e + n + 2 * NW * K - 1) // (2 * NW * K)) * (2 * NW * K)
  loop_idx = jnp.arange(n, dtype=jnp.int32)
  pad = jnp.full((etot - e - n,), N_NODES, jnp.int32)
  src3 = jnp.concatenate([ei[0], loop_idx, pad]).reshape(NW, -1, K)
  dst3 = jnp.concatenate([ei[1], loop_idx, pad]).reshape(NW, -1, K)

  x_pad = jnp.zeros((NPAD, DIM), jnp.float32).at[:n].set(x)
  zeros = jnp.zeros((NPAD, DIM), jnp.float32)

  h1, s1, d1, m1 = _tc_lin(x_pad, W1, att_src1, att_dst1)
  mvec1 = jnp.full((L,), m1[0, 0], jnp.float32)
  acc1, den1 = _sc_edge(h1, s1.reshape(NPAD), d1.reshape(NPAD), src3, dst3,
                        mvec1, zeros)

  h2, s2, d2, m2 = _tc_mid(acc1[0], acc1[1], den1, b1, W2,
                           att_src2, att_dst2)
  mvec2 = jnp.full((L,), m2[0, 0], jnp.float32)
  acc2, den2 = _sc_edge(h2, s2.reshape(NPAD), d2.reshape(NPAD), src3, dst3,
                        mvec2, zeros)

  return _tc_pool(acc2[0], acc2[1], den2, b2, batch.astype(jnp.int32))


# speedup vs baseline: 17.2843x; 1.0000x over previous
import functools
import jax
import jax.numpy as jnp
from jax import lax
from jax.experimental import pallas as pl
from jax.experimental.pallas import tpu as pltpu
from jax.experimental.pallas import tpu_sc as plsc

N_NODES = 10000
DIM = 128
NUM_GRAPHS = 64

NC, NS, L = 2, 16, 16
NW = NC * NS
NPAD = 10112
K = 48
RPT = NPAD // NS


def _leaky(x, slope):
  return jnp.where(x > 0, x, x * slope)


def _lin_body(x_ref, w_ref, as_ref, ad_ref, h_ref, s_ref, d_ref, m_ref):
  x = x_ref[...]
  h = lax.dot_general(x, w_ref[...], (((1,), (1,)), ((), ())),
                      preferred_element_type=jnp.float32)
  h_ref[...] = h
  s = lax.dot_general(h, as_ref[...], (((1,), (0,)), ((), ())),
                      preferred_element_type=jnp.float32)
  d = lax.dot_general(h, ad_ref[...], (((1,), (0,)), ((), ())),
                      preferred_element_type=jnp.float32)
  rows = lax.broadcasted_iota(jnp.int32, (NPAD, 1), 0)
  s = jnp.where(rows < N_NODES, s, 0.0)
  d = jnp.where(rows < N_NODES, d, 0.0)
  s_ref[...] = s
  d_ref[...] = d
  m_ref[...] = _leaky(jnp.max(s) + jnp.max(d), 0.2).reshape(1, 1)


def _tc_lin(x, w, att_s, att_d):
  return pl.pallas_call(
      _lin_body,
      out_shape=[
          jax.ShapeDtypeStruct((NPAD, DIM), jnp.float32),
          jax.ShapeDtypeStruct((NPAD, 1), jnp.float32),
          jax.ShapeDtypeStruct((NPAD, 1), jnp.float32),
          jax.ShapeDtypeStruct((1, 1), jnp.float32),
      ],
  )(x, w, att_s.reshape(DIM, 1), att_d.reshape(DIM, 1))


def _mid_body(a0_ref, a1_ref, den_ref, b_ref, w_ref, as_ref, ad_ref,
              h_ref, s_ref, d_ref, m_ref):
  num = a0_ref[...] + a1_ref[...]
  den = jnp.sum(den_ref[...], axis=0).reshape(NPAD, 1)
  o = num / (den + 1e-16) + b_ref[...]
  o = _leaky(o, 0.01)
  h = lax.dot_general(o, w_ref[...], (((1,), (1,)), ((), ())),
                      preferred_element_type=jnp.float32)
  h = jnp.where(lax.broadcasted_iota(jnp.int32, (NPAD, 1), 0) < N_NODES,
                h, 0.0)
  h_ref[...] = h
  s = lax.dot_general(h, as_ref[...], (((1,), (0,)), ((), ())),
                      preferred_element_type=jnp.float32)
  d = lax.dot_general(h, ad_ref[...], (((1,), (0,)), ((), ())),
                      preferred_element_type=jnp.float32)
  rows = lax.broadcasted_iota(jnp.int32, (NPAD, 1), 0)
  s = jnp.where(rows < N_NODES, s, 0.0)
  d = jnp.where(rows < N_NODES, d, 0.0)
  s_ref[...] = s
  d_ref[...] = d
  m_ref[...] = _leaky(jnp.max(s) + jnp.max(d), 0.2).reshape(1, 1)


def _tc_mid(a0, a1, den, b, w, att_s, att_d):
  return pl.pallas_call(
      _mid_body,
      out_shape=[
          jax.ShapeDtypeStruct((NPAD, DIM), jnp.float32),
          jax.ShapeDtypeStruct((NPAD, 1), jnp.float32),
          jax.ShapeDtypeStruct((NPAD, 1), jnp.float32),
          jax.ShapeDtypeStruct((1, 1), jnp.float32),
      ],
  )(a0, a1, den, b.reshape(1, DIM), w,
    att_s.reshape(DIM, 1), att_d.reshape(DIM, 1))


def _pool_body(a0_ref, a1_ref, den_ref, b_ref, batch_ref, out_ref):
  num = a0_ref[...] + a1_ref[...]
  den = jnp.sum(den_ref[...], axis=0).reshape(NPAD, 1)
  o = num / (den + 1e-16) + b_ref[...]
  o = o[:N_NODES]
  bb = batch_ref[...]
  iot = lax.broadcasted_iota(jnp.int32, (NUM_GRAPHS, N_NODES), 0)
  onehot = (iot == bb).astype(jnp.float32)
  out_ref[...] = lax.dot_general(onehot, o, (((1,), (0,)), ((), ())),
                                 preferred_element_type=jnp.float32)


def _tc_pool(a0, a1, den, b, batch):
  return pl.pallas_call(
      _pool_body,
      out_shape=jax.ShapeDtypeStruct((NUM_GRAPHS, DIM), jnp.float32),
  )(a0, a1, den, b.reshape(1, DIM), batch.reshape(1, N_NODES))


def _sc_edge_body(nch,
                  h_hbm, asrc_hbm, adst_hbm, src_hbm, dst_hbm, mvec_hbm,
                  zeros_hbm, acc_out, den_out,
                  acc_s, asrc_v, adst_v, den_v,
                  srcc0_v, dstc0_v, srcc1_v, dstc1_v, g0_v, g1_v, m_v):
  cid = lax.axis_index("c")
  sid = lax.axis_index("s")
  wid = sid * NC + cid

  rbase = sid * RPT
  pltpu.sync_copy(zeros_hbm.at[pl.ds(rbase, RPT)],
                  acc_s.at[pl.ds(rbase, RPT)])

  @pl.loop(0, NPAD // L)
  def _z_loop(c):
    den_v[pl.ds(c * L, L)] = jnp.zeros((L,), jnp.float32)

  pltpu.sync_copy(asrc_hbm, asrc_v)
  pltpu.sync_copy(adst_hbm, adst_v)
  pltpu.sync_copy(mvec_hbm, m_v)

  plsc.subcore_barrier()

  def _chunk(c, srcc_v, dstc_v, g_v):
    pltpu.sync_copy(src_hbm.at[wid, c], srcc_v)
    pltpu.sync_copy(dst_hbm.at[wid, c], dstc_v)
    pltpu.sync_copy(h_hbm.at[srcc_v], g_v)
    mv = m_v[...]
    iot = lax.iota(jnp.int32, L)
    for l in range(K // L):
      si = srcc_v[pl.ds(l * L, L)]
      di = dstc_v[pl.ds(l * L, L)]
      a = plsc.load_gather(asrc_v, [si]) + plsc.load_gather(adst_v, [di])
      p = jnp.exp(_leaky(a, 0.2) - mv)
      for i in range(L):
        plsc.addupdate_scatter(den_v, [di], p, mask=iot == i)
      for i in range(L):
        e = l * L + i
        pb = lax.gather(
            p, jnp.full((L, 1), i, jnp.int32),
            lax.GatherDimensionNumbers(offset_dims=(),
                                       collapsed_slice_dims=(0,),
                                       start_index_map=(0,)),
            (1,), mode=lax.GatherScatterMode.PROMISE_IN_BOUNDS)
        for j in range(DIM // L):
          g_v[e, pl.ds(j * L, L)] = g_v[e, pl.ds(j * L, L)] * pb
    pltpu.sync_copy(g_v, acc_s.at[dstc_v], add=True)

  @pl.loop(0, nch // 2)
  def _e_loop(cc):
    _chunk(cc * 2, srcc0_v, dstc0_v, g0_v)
    _chunk(cc * 2 + 1, srcc1_v, dstc1_v, g1_v)

  plsc.subcore_barrier()

  pltpu.sync_copy(acc_s.at[pl.ds(rbase, RPT)],
                  acc_out.at[cid, pl.ds(rbase, RPT)])
  pltpu.sync_copy(den_v, den_out.at[wid])


def _sc_edge(h, asrc, adst, src3, dst3, mvec, zeros):
  nch = src3.shape[1]
  kern = pl.kernel(
      functools.partial(_sc_edge_body, nch),
      out_type=[
          jax.ShapeDtypeStruct((NC, NPAD, DIM), jnp.float32),
          jax.ShapeDtypeStruct((NW, NPAD), jnp.float32),
      ],
      mesh=plsc.VectorSubcoreMesh(core_axis_name="c", subcore_axis_name="s",
                                  num_cores=NC, num_subcores=NS),
      compiler_params=pltpu.CompilerParams(needs_layout_passes=False),
      scratch_types=[
          pltpu.VMEM_SHARED((NPAD, DIM), jnp.float32),
          pltpu.VMEM((NPAD,), jnp.float32),
          pltpu.VMEM((NPAD,), jnp.float32),
          pltpu.VMEM((NPAD,), jnp.float32),
          pltpu.VMEM((K,), jnp.int32),
          pltpu.VMEM((K,), jnp.int32),
          pltpu.VMEM((K,), jnp.int32),
          pltpu.VMEM((K,), jnp.int32),
          pltpu.VMEM((K, DIM), jnp.float32),
          pltpu.VMEM((K, DIM), jnp.float32),
          pltpu.VMEM((L,), jnp.float32),
      ],
  )
  return kern(h, asrc, adst, src3, dst3, mvec, zeros)


def kernel(x, edge_index, batch, W1, att_src1, att_dst1, b1,
           W2, att_src2, att_dst2, b2):
  n = x.shape[0]
  e = edge_index.shape[1]
  ei = edge_index.astype(jnp.int32)

  etot = ((e + n + 2 * NW * K - 1) // (2 * NW * K)) * (2 * NW * K)
  loop_idx = jnp.arange(n, dtype=jnp.int32)
  pad = jnp.full((etot - e - n,), N_NODES, jnp.int32)
  src3 = jnp.concatenate([ei[0], loop_idx, pad]).reshape(NW, -1, K)
  dst3 = jnp.concatenate([ei[1], loop_idx, pad]).reshape(NW, -1, K)

  x_pad = jnp.zeros((NPAD, DIM), jnp.float32).at[:n].set(x)
  zeros = jnp.zeros((NPAD, DIM), jnp.float32)

  h1, s1, d1, m1 = _tc_lin(x_pad, W1, att_src1, att_dst1)
  mvec1 = jnp.full((L,), m1[0, 0], jnp.float32)
  acc1, den1 = _sc_edge(h1, s1.reshape(NPAD), d1.reshape(NPAD), src3, dst3,
                        mvec1, zeros)

  h2, s2, d2, m2 = _tc_mid(acc1[0], acc1[1], den1, b1, W2,
                           att_src2, att_dst2)
  mvec2 = jnp.full((L,), m2[0, 0], jnp.float32)
  acc2, den2 = _sc_edge(h2, s2.reshape(NPAD), d2.reshape(NPAD), src3, dst3,
                        mvec2, zeros)

  return _tc_pool(acc2[0], acc2[1], den2, b2, batch.astype(jnp.int32))
